# baseline (speedup 1.0000x reference)
import functools

import jax
import jax.numpy as jnp
from jax import lax
from jax.experimental import pallas as pl
from jax.experimental.pallas import tpu as pltpu
from jax.experimental.pallas import tpu_sc as plsc

B = 16384
DA = 64
DOUT = 128
V = 100000
NC = 2
NS = 16
NW = NC * NS
B_PER_W = B // NW
CHUNK = 128
NCHUNK = B_PER_W // CHUNK

_RT = 25088


def _proj_body_bias(t_ref, w_ref, b_ref, out_ref):
    w = w_ref[...][:, :DA].astype(jnp.bfloat16)
    out_ref[...] = lax.dot_general(
        t_ref[...].astype(jnp.bfloat16), w, (((0,), (1,)), ((), ())),
        preferred_element_type=jnp.float32) + b_ref[...]


def _proj_body_nobias(t_ref, w_ref, out_ref):
    w = w_ref[...][:, DA:].astype(jnp.bfloat16)
    out_ref[...] = lax.dot_general(
        t_ref[...].astype(jnp.bfloat16), w, (((0,), (1,)), ((), ())),
        preferred_element_type=jnp.float32)


def _tc_project(table_t, w_full, b2d=None):
    grid = (pl.cdiv(V, _RT),)
    specs = [
        pl.BlockSpec((DA, _RT), lambda i: (0, i)),
        pl.BlockSpec((DOUT, DOUT), lambda i: (0, 0)),
    ]
    args = [table_t, w_full]
    body = _proj_body_nobias
    if b2d is not None:
        specs.append(pl.BlockSpec((1, DOUT), lambda i: (0, 0)))
        args.append(b2d)
        body = _proj_body_bias
    return pl.pallas_call(
        body,
        grid=grid,
        compiler_params=pltpu.CompilerParams(
            fuse_transposed_lhs_in_matmul=True),
        in_specs=specs,
        out_specs=pl.BlockSpec((_RT, DOUT), lambda i: (i, 0)),
        out_shape=jax.ShapeDtypeStruct((V, DOUT), jnp.float32),
    )(*args)


@functools.partial(
    pl.kernel,
    mesh=plsc.VectorSubcoreMesh(core_axis_name="c", subcore_axis_name="s"),
    out_type=jax.ShapeDtypeStruct((B, DOUT), jnp.float32),
    scratch_types=[
        pltpu.VMEM((B_PER_W,), jnp.int32),
        pltpu.VMEM((B_PER_W, DOUT), jnp.float32),
        pltpu.SemaphoreType.DMA,
    ],
)
def _sc_gather(idx_hbm, t_hbm, out_hbm, idx_v, rows_v, sem):
    wid = lax.axis_index("s") * NC + lax.axis_index("c")
    base = wid * B_PER_W
    pltpu.sync_copy(idx_hbm.at[pl.ds(base, B_PER_W)], idx_v)
    copies = []
    for j in range(NCHUNK):
        copies.append(pltpu.async_copy(
            t_hbm.at[idx_v.at[pl.ds(j * CHUNK, CHUNK)]],
            rows_v.at[pl.ds(j * CHUNK, CHUNK)], sem))
    for c in copies:
        c.wait()
    pltpu.sync_copy(rows_v, out_hbm.at[pl.ds(base, B_PER_W)])


_BT = 8192


def _addrelu_body(a_ref, m_ref, out_ref):
    out_ref[...] = jnp.maximum(a_ref[...] + m_ref[...], 0.0)


def _tc_addrelu(ga, gm):
    return pl.pallas_call(
        _addrelu_body,
        grid=(B // _BT,),
        in_specs=[
            pl.BlockSpec((_BT, DOUT), lambda i: (i, 0)),
            pl.BlockSpec((_BT, DOUT), lambda i: (i, 0)),
        ],
        out_specs=pl.BlockSpec((_BT, DOUT), lambda i: (i, 0)),
        out_shape=jax.ShapeDtypeStruct((B, DOUT), jnp.float32),
    )(ga, gm)


def kernel(aff_idx, mat_idx, aff_table, mat_table, W, b):
    ai = aff_idx.astype(jnp.int32)
    mi = mat_idx.astype(jnp.int32)
    b2d = b.reshape(1, DOUT)
    proj_aff = _tc_project(aff_table.T, W, b2d)
    ga = _sc_gather(ai, proj_aff)
    proj_mat = _tc_project(mat_table.T, W)
    gm = _sc_gather(mi, proj_mat)
    return _tc_addrelu(ga, gm)


# speedup vs baseline: 1.0570x; 1.0570x over previous
import functools

import jax
import jax.numpy as jnp
from jax import lax
from jax.experimental import pallas as pl
from jax.experimental.pallas import tpu as pltpu
from jax.experimental.pallas import tpu_sc as plsc

B = 16384
DA = 64
DOUT = 128
V = 100000
H = 50176
NC = 2
NS = 16
NW = NC * NS
B_PER_W = B // NW
CHUNK = 128
NCHUNK = B_PER_W // CHUNK

_RT = 12544


def _pack16(lo_f32, hi_f32):
    lo = lax.bitcast_convert_type(
        lo_f32.astype(jnp.bfloat16), jnp.uint16).astype(jnp.uint32)
    hi = lax.bitcast_convert_type(
        hi_f32.astype(jnp.bfloat16), jnp.uint16).astype(jnp.uint32)
    return lax.bitcast_convert_type(lo | (hi << 16), jnp.float32)


def _proj_body_bias(t_lo_ref, t_hi_ref, w_ref, b_ref, out_ref):
    w = w_ref[...][:, :DA].astype(jnp.bfloat16)
    dn = (((0,), (1,)), ((), ()))
    lo = lax.dot_general(t_lo_ref[...].astype(jnp.bfloat16), w, dn,
                         preferred_element_type=jnp.float32) + b_ref[...]
    hi = lax.dot_general(t_hi_ref[...].astype(jnp.bfloat16), w, dn,
                         preferred_element_type=jnp.float32) + b_ref[...]
    out_ref[...] = _pack16(lo, hi)


def _proj_body_nobias(t_lo_ref, t_hi_ref, w_ref, out_ref):
    w = w_ref[...][:, DA:].astype(jnp.bfloat16)
    dn = (((0,), (1,)), ((), ()))
    lo = lax.dot_general(t_lo_ref[...].astype(jnp.bfloat16), w, dn,
                         preferred_element_type=jnp.float32)
    hi = lax.dot_general(t_hi_ref[...].astype(jnp.bfloat16), w, dn,
                         preferred_element_type=jnp.float32)
    out_ref[...] = _pack16(lo, hi)


def _tc_project(table_t, w_full, b2d=None):
    grid = (H // _RT,)
    specs = [
        pl.BlockSpec((DA, _RT), lambda i: (0, i)),
        pl.BlockSpec((DA, _RT), lambda i: (0, i + H // _RT)),
        pl.BlockSpec((DOUT, DOUT), lambda i: (0, 0)),
    ]
    args = [table_t, table_t, w_full]
    body = _proj_body_nobias
    if b2d is not None:
        specs.append(pl.BlockSpec((1, DOUT), lambda i: (0, 0)))
        args.append(b2d)
        body = _proj_body_bias
    return pl.pallas_call(
        body,
        grid=grid,
        compiler_params=pltpu.CompilerParams(
            fuse_transposed_lhs_in_matmul=True),
        in_specs=specs,
        out_specs=pl.BlockSpec((_RT, DOUT), lambda i: (i, 0)),
        out_shape=jax.ShapeDtypeStruct((H, DOUT), jnp.float32),
    )(*args)


@functools.partial(
    pl.kernel,
    mesh=plsc.VectorSubcoreMesh(core_axis_name="c", subcore_axis_name="s"),
    out_type=jax.ShapeDtypeStruct((B, DOUT), jnp.float32),
    scratch_types=[
        pltpu.VMEM((B_PER_W,), jnp.int32),
        pltpu.VMEM((B_PER_W, DOUT), jnp.float32),
        pltpu.SemaphoreType.DMA,
    ],
)
def _sc_gather(idx_hbm, t_hbm, out_hbm, idx_v, rows_v, sem):
    wid = lax.axis_index("s") * NC + lax.axis_index("c")
    base = wid * B_PER_W
    pltpu.sync_copy(idx_hbm.at[pl.ds(base, B_PER_W)], idx_v)
    copies = []
    for j in range(NCHUNK):
        copies.append(pltpu.async_copy(
            t_hbm.at[idx_v.at[pl.ds(j * CHUNK, CHUNK)]],
            rows_v.at[pl.ds(j * CHUNK, CHUNK)], sem))
    for c in copies:
        c.wait()
    pltpu.sync_copy(rows_v, out_hbm.at[pl.ds(base, B_PER_W)])


_BT = 8192


def _unpack16(packed_f32, shift):
    w = lax.bitcast_convert_type(packed_f32, jnp.uint32)
    half = lax.shift_right_logical(w, shift) & jnp.uint32(0xFFFF)
    return lax.bitcast_convert_type(
        half.astype(jnp.uint16), jnp.bfloat16).astype(jnp.float32)


def _addrelu_body(a_ref, m_ref, sa_ref, sm_ref, out_ref):
    va = _unpack16(a_ref[...], sa_ref[...].astype(jnp.uint32))
    vm = _unpack16(m_ref[...], sm_ref[...].astype(jnp.uint32))
    out_ref[...] = jnp.maximum(va + vm, 0.0)


def _tc_addrelu(ga, gm, sa, sm):
    return pl.pallas_call(
        _addrelu_body,
        grid=(B // _BT,),
        in_specs=[
            pl.BlockSpec((_BT, DOUT), lambda i: (i, 0)),
            pl.BlockSpec((_BT, DOUT), lambda i: (i, 0)),
            pl.BlockSpec((_BT, 1), lambda i: (i, 0)),
            pl.BlockSpec((_BT, 1), lambda i: (i, 0)),
        ],
        out_specs=pl.BlockSpec((_BT, DOUT), lambda i: (i, 0)),
        out_shape=jax.ShapeDtypeStruct((B, DOUT), jnp.float32),
    )(ga, gm, sa, sm)


def kernel(aff_idx, mat_idx, aff_table, mat_table, W, b):
    ai = aff_idx.astype(jnp.int32)
    mi = mat_idx.astype(jnp.int32)
    b2d = b.reshape(1, DOUT)
    sa = ((ai >= H) * 16).astype(jnp.int8).reshape(B, 1)
    sm = ((mi >= H) * 16).astype(jnp.int8).reshape(B, 1)
    proj_aff = _tc_project(aff_table.T, W, b2d)
    ga = _sc_gather(ai % H, proj_aff)
    proj_mat = _tc_project(mat_table.T, W)
    gm = _sc_gather(mi % H, proj_mat)
    return _tc_addrelu(ga, gm, sa, sm)
